# baseline (speedup 1.0000x reference)
import functools

import jax
import jax.numpy as jnp
from jax import lax
from jax.experimental import pallas as pl
from jax.experimental.pallas import tpu as pltpu
from jax.experimental.pallas import tpu_sc as plsc

_NC = 2
_NS = 16
_NW = _NC * _NS
_SUP = 16
_BATCH = 64


def _gather3_body(B, tabs,
                  t0, i0, t1, i1, t2, i2, tl0, tl1, tl2,
                  o0, o1, o2,
                  idxb, selb, cbuf, ext, jbig,
                  sem_i, sem_c, sem_s, sem_o):
    wid = lax.axis_index("s") * _NC + lax.axis_index("c")
    padrow = B + wid
    iota = lax.iota(jnp.int32, 16)

    def drain(o):
        pltpu.make_async_copy(
            ext.at[0], o.at[plsc.Indices(jbig.at[0])], sem_s).wait()

    def flush_blk(o, bf, pend):
        pm_b = lax.rem(lax.div(bf, _BATCH), 2)
        bfo = lax.rem(bf, _BATCH)

        def pad(g, _):
            rows = g * 16 + iota
            m = jnp.logical_and(rows >= bfo, rows < _BATCH)
            plsc.store_scatter(jbig, [jnp.full((16,), pm_b, jnp.int32),
                                      rows],
                               jnp.full((16,), padrow, jnp.int32), mask=m)
            return _

        lax.fori_loop(0, _BATCH // 16, pad, 0)
        pltpu.async_copy(ext.at[pm_b], o.at[plsc.Indices(jbig.at[pm_b])],
                         sem_s)

        @pl.when(pend == 1)
        def _():
            drain(o)

    def group_step(o, bf, pend, m, jv, load_vals):
        need = lax.rem(bf, _BATCH) > (_BATCH - 16)

        @pl.when(need)
        def _():
            flush_blk(o, bf, pend)

        bf = jnp.where(need, (lax.div(bf, _BATCH) + 1) * _BATCH, bf)
        pend = jnp.where(need, 1, pend)

        pm_b = lax.rem(lax.div(bf, _BATCH), 2)
        bfo = lax.rem(bf, _BATCH)
        mi = m.astype(jnp.int32)
        rank = plsc.cumsum(mi) - mi
        rows = bfo + rank
        pmbv = jnp.full((16,), pm_b, jnp.int32)
        for c, val in load_vals():
            plsc.store_scatter(ext, [pmbv, rows,
                                     jnp.full((16,), c, jnp.int32)],
                               val, mask=m)
        plsc.store_scatter(jbig, [pmbv, rows], jv, mask=m)
        cnt = jnp.max(plsc.all_reduce_population_count(m))
        bf2 = bf + cnt
        full = jnp.logical_and(lax.rem(bf2, _BATCH) == 0, cnt > 0)

        @pl.when(full)
        def _(pm_b=pm_b):
            pltpu.async_copy(ext.at[pm_b],
                             o.at[plsc.Indices(jbig.at[pm_b])], sem_s)

            @pl.when(pend == 1)
            def _():
                drain(o)

        pend = jnp.where(full, 1, pend)
        return bf2, pend

    for (t, i, o, tl, d, V, cw, npw, nfull, tw, towner) in (
            (t0, i0, o0, tl0) + tabs[0],
            (t1, i1, o1, tl1) + tabs[1],
            (t2, i2, o2, tl2) + tabs[2]):
        pltpu.async_copy(i, idxb, sem_i).wait()
        lo_w = wid * (npw * cw)
        hi_w = jnp.minimum(V, lo_w + npw * cw)

        def select(q, pos, lo_w=lo_w, hi_w=hi_w):
            r = idxb[pl.ds(q * 16, 16)]
            m = jnp.logical_and(r >= lo_w, r < hi_w)
            packed = lax.shift_left(r - lo_w, 14) + (iota + q * 16)
            plsc.store_compressed(selb.at[pl.ds(pos, 16)], packed, mask=m)
            return pos + jnp.max(plsc.all_reduce_population_count(m))

        pos = lax.fori_loop(0, B // 16, select, jnp.int32(0), unroll=2)

        nch = jnp.minimum(jnp.maximum(nfull - wid * npw, 0), npw)

        def fire(ci, pm, t=t, d=d, cw=cw):
            start = pl.multiple_of(ci * cw, 128)
            return pltpu.async_copy(
                t.at[:, pl.ds(start, cw)],
                cbuf.at[pm, pl.ds(0, d), pl.ds(0, cw)], sem_c)

        def wait_chunk(t=t, d=d, cw=cw):
            pltpu.make_async_copy(
                t.at[:, pl.ds(0, cw)],
                cbuf.at[0, pl.ds(0, d), pl.ds(0, cw)], sem_c).wait()

        @pl.when(nch > 0)
        def _():
            fire(wid * npw, 0)

        @pl.when(nch > 1)
        def _():
            fire(wid * npw + 1, 1)

        def super_loop(s, carry, lo_w=lo_w, pos=pos, o=o, d=d, cw=cw,
                       npw=npw, nch=nch):
            sup_lo = s * (_SUP * cw)
            sup_hi = sup_lo + _SUP * cw

            def rescan(q, n, sup_lo=sup_lo, sup_hi=sup_hi, pos=pos):
                e = selb[pl.ds(q * 16, 16)]
                rl = lax.shift_right_logical(e, 14)
                m = jnp.logical_and(rl >= sup_lo, rl < sup_hi)
                m = jnp.logical_and(m, (q * 16 + iota) < pos)
                plsc.store_compressed(idxb.at[pl.ds(n, 16)], e, mask=m)
                return n + jnp.max(plsc.all_reduce_population_count(m))

            sup_n = lax.cond(
                s * _SUP < nch,
                lambda _: lax.fori_loop(0, lax.div(pos + 15, 16), rescan,
                                        jnp.int32(0)),
                lambda _: jnp.int32(0), 0)

            def chunk_loop(k, carry2, s=s, sup_n=sup_n, npw=npw, nch=nch,
                           o=o, cw=cw, d=d):
                il = s * _SUP + k
                ci = wid * npw + il

                @pl.when(il + 2 < nch)
                def _(il=il, ci=ci):
                    fire(ci + 2, lax.rem(il + 2, 3))

                def do(carry3, il=il, ci=ci, o=o, cw=cw, sup_n=sup_n, d=d):
                    wait_chunk()
                    pm = lax.rem(il, 3)
                    lo_c = il * cw

                    def group(g, c4, pm=pm, lo_c=lo_c, sup_n=sup_n, o=o,
                              cw=cw, d=d):
                        bf, pend = c4
                        e = idxb[pl.ds(g * 16, 16)]
                        rl = lax.shift_right_logical(e, 14)
                        j = lax.bitwise_and(e, 16383)
                        m = jnp.logical_and(rl >= lo_c, rl < lo_c + cw)
                        m = jnp.logical_and(m, (g * 16 + iota) < sup_n)
                        rv = jnp.where(m, rl - lo_c, 0)
                        jv = jnp.where(m, j, padrow)

                        def load_vals(rv=rv, pm=pm, d=d):
                            for c in range(d):
                                yield c, plsc.load_gather(
                                    cbuf,
                                    [jnp.full((16,), pm, jnp.int32),
                                     jnp.full((16,), c, jnp.int32), rv])

                        return group_step(o, bf, pend, m, jv, load_vals)

                    return lax.fori_loop(0, lax.div(sup_n + 15, 16),
                                         group, carry3)

                return lax.cond(il < nch, do, lambda c: c, carry2)

            return lax.fori_loop(0, _SUP, chunk_loop, carry)

        nsup = -(-npw // _SUP)
        carry = lax.fori_loop(0, nsup, super_loop,
                              (jnp.int32(0), jnp.int32(0)))
        bf, pend = carry

        if tw > 0:
            @pl.when(wid == towner)
            def _(tl=tl, o=o, d=d, tw=tw, bf=bf, pend=pend, pos=pos,
                  nfull=nfull, cw=cw, lo_w=lo_w):
                pltpu.async_copy(tl, cbuf.at[0, pl.ds(0, tw), pl.ds(0, 128)],
                                 sem_c).wait()
                lo_c = nfull * cw - lo_w

                def rescan(q, n, lo_c=lo_c, tw=tw, pos=pos):
                    e = selb[pl.ds(q * 16, 16)]
                    rl = lax.shift_right_logical(e, 14)
                    m = jnp.logical_and(rl >= lo_c, rl < lo_c + tw)
                    m = jnp.logical_and(m, (q * 16 + iota) < pos)
                    plsc.store_compressed(idxb.at[pl.ds(n, 16)], e, mask=m)
                    return n + jnp.max(plsc.all_reduce_population_count(m))

                n = lax.fori_loop(0, lax.div(pos + 15, 16), rescan,
                                  jnp.int32(0))

                def group(g, c4, n=n, o=o, d=d, lo_c=lo_c):
                    bf2, pend2 = c4
                    e = idxb[pl.ds(g * 16, 16)]
                    rl = lax.shift_right_logical(e, 14)
                    j = lax.bitwise_and(e, 16383)
                    m = (g * 16 + iota) < n
                    rv = jnp.where(m, rl - lo_c, 0)
                    jv = jnp.where(m, j, padrow)

                    def load_vals(rv=rv, d=d):
                        for c in range(d):
                            yield c, plsc.load_gather(
                                cbuf, [jnp.zeros((16,), jnp.int32), rv,
                                       jnp.full((16,), c, jnp.int32)])

                    return group_step(o, bf2, pend2, m, jv, load_vals)

                bf_t, pend_t = lax.fori_loop(0, lax.div(n + 15, 16),
                                             group, (bf, pend))
                _final(o, bf_t, pend_t, flush_blk, drain)

            @pl.when(wid != towner)
            def _(o=o, bf=bf, pend=pend):
                _final(o, bf, pend, flush_blk, drain)
        else:
            _final(o, bf, pend, flush_blk, drain)


def _final(o, bf, pend, flush_blk, drain):
    bfo = lax.rem(bf, _BATCH)

    @pl.when(bfo > 0)
    def _():
        flush_blk(o, bf, pend)
        drain(o)

    @pl.when(jnp.logical_and(bfo == 0, pend == 1))
    def _():
        drain(o)


def _chunk_plan(V, cw):
    nfull = V // cw
    tw = V - nfull * cw
    npw = -(-nfull // _NW)
    towner = nfull // npw if tw > 0 else 0
    return cw, npw, nfull, tw, towner


def _make_gather3(B, d0, V0, d1, V1, d2, V2):
    tabs = tuple((d, V) + _chunk_plan(V, cw)
                 for d, V, cw in ((d0, V0, 384), (d1, V1, 384),
                                  (d2, V2, 128)))
    max_cw = max(t[2] for t in tabs)
    mesh = plsc.VectorSubcoreMesh(core_axis_name="c", subcore_axis_name="s")
    return pl.kernel(
        functools.partial(_gather3_body, B, tabs),
        out_type=(
            jax.ShapeDtypeStruct((B + _NW, 128), jnp.float32),
            jax.ShapeDtypeStruct((B + _NW, 128), jnp.float32),
            jax.ShapeDtypeStruct((B + _NW, 128), jnp.float32),
        ),
        mesh=mesh,
        scratch_types=[
            pltpu.VMEM((B,), jnp.int32),
            pltpu.VMEM((B,), jnp.int32),
            pltpu.VMEM((3, 64, max_cw), jnp.float32),
            pltpu.VMEM((2, _BATCH, 128), jnp.float32),
            pltpu.VMEM((2, _BATCH), jnp.int32),
            pltpu.SemaphoreType.DMA,
            pltpu.SemaphoreType.DMA,
            pltpu.SemaphoreType.DMA,
            pltpu.SemaphoreType.DMA,
        ],
        compiler_params=pltpu.CompilerParams(needs_layout_passes=False),
    )


def _mlp_body(x_ref, w_ref, b_ref, o_ref):
    x = jnp.maximum(x_ref[...], 0.0)
    acc = jax.lax.dot_general(
        x, w_ref[...], (((1,), (0,)), ((), ())),
        preferred_element_type=jnp.float32)
    o_ref[...] = jnp.maximum(acc + b_ref[...], 0.0)


def _mlp(x, w, b):
    B, K = x.shape
    N = w.shape[1]
    BLK = 2048
    return pl.pallas_call(
        _mlp_body,
        grid=(B // BLK,),
        in_specs=[
            pl.BlockSpec((BLK, K), lambda i: (i, 0)),
            pl.BlockSpec((K, N), lambda i: (0, 0)),
            pl.BlockSpec((N,), lambda i: (0,)),
        ],
        out_specs=pl.BlockSpec((BLK, N), lambda i: (i, 0)),
        out_shape=jax.ShapeDtypeStruct((B, N), jnp.float32),
    )(x, w, b)


def _tail_pad(x, cw):
    nfull = x.shape[0] // cw
    d = x.shape[1]
    tail = x[nfull * cw:, :]
    return jnp.pad(tail, ((0, 0), (0, 128 - d)))


def kernel(arg0_1, arg1_1, arg2_1, arg3_1, arg4_1, arg5_1, arg6_1, arg7_1, arg8_1):
    B = arg1_1.shape[0]
    d0, d1, d2 = arg0_1.shape[1], arg2_1.shape[1], arg4_1.shape[1]
    g = _make_gather3(B, d0, arg0_1.shape[0], d1, arg2_1.shape[0],
                      d2, arg4_1.shape[0])
    f0, f1, f2 = g(arg0_1.T, arg1_1, arg2_1.T, arg3_1, arg4_1.T, arg5_1,
                   _tail_pad(arg0_1, 384), _tail_pad(arg2_1, 384),
                   _tail_pad(arg4_1, 128))
    relu_1 = _mlp(arg7_1, arg6_1, arg8_1)
    return (f0[:B, :d0], f1[:B, :d1], f2[:B, :d2], arg6_1, relu_1)


# speedup vs baseline: 1.4088x; 1.4088x over previous
import functools

import jax
import jax.numpy as jnp
from jax import lax
from jax.experimental import pallas as pl
from jax.experimental.pallas import tpu as pltpu
from jax.experimental.pallas import tpu_sc as plsc

_NC = 2
_NS = 16
_NW = _NC * _NS
_SUP = 4
_BATCH = 64


def _gather3_body(B, tabs,
                  t0, i0, t1, i1, t2, i2, tl0, tl1, tl2,
                  o0, o1, o2,
                  idxb, selb, cbuf, ext, jbig,
                  sem_i, sem_c, sem_s, sem_o):
    wid = lax.axis_index("s") * _NC + lax.axis_index("c")
    padrow = B + wid
    iota = lax.iota(jnp.int32, 16)

    def drain(o):
        pltpu.make_async_copy(
            ext.at[0], o.at[plsc.Indices(jbig.at[0])], sem_s).wait()

    def flush_blk(o, bf, pend):
        pm_b = lax.rem(lax.div(bf, _BATCH), 2)
        bfo = lax.rem(bf, _BATCH)

        def pad(g, _):
            rows = g * 16 + iota
            m = jnp.logical_and(rows >= bfo, rows < _BATCH)
            plsc.store_scatter(jbig, [jnp.full((16,), pm_b, jnp.int32),
                                      rows],
                               jnp.full((16,), padrow, jnp.int32), mask=m)
            return _

        lax.fori_loop(0, _BATCH // 16, pad, 0)
        pltpu.async_copy(ext.at[pm_b], o.at[plsc.Indices(jbig.at[pm_b])],
                         sem_s)

        @pl.when(pend == 1)
        def _():
            drain(o)

    def group_step(o, bf, pend, m, jv, load_vals):
        need = lax.rem(bf, _BATCH) > (_BATCH - 16)

        @pl.when(need)
        def _():
            flush_blk(o, bf, pend)

        bf = jnp.where(need, (lax.div(bf, _BATCH) + 1) * _BATCH, bf)
        pend = jnp.where(need, 1, pend)

        pm_b = lax.rem(lax.div(bf, _BATCH), 2)
        bfo = lax.rem(bf, _BATCH)
        mi = m.astype(jnp.int32)
        rank = plsc.cumsum(mi) - mi
        rows = bfo + rank
        pmbv = jnp.full((16,), pm_b, jnp.int32)
        for c, val in load_vals():
            plsc.store_scatter(ext, [pmbv, rows,
                                     jnp.full((16,), c, jnp.int32)],
                               val, mask=m)
        plsc.store_scatter(jbig, [pmbv, rows], jv, mask=m)
        cnt = jnp.max(plsc.all_reduce_population_count(m))
        bf2 = bf + cnt
        full = jnp.logical_and(lax.rem(bf2, _BATCH) == 0, cnt > 0)

        @pl.when(full)
        def _(pm_b=pm_b):
            pltpu.async_copy(ext.at[pm_b],
                             o.at[plsc.Indices(jbig.at[pm_b])], sem_s)

            @pl.when(pend == 1)
            def _():
                drain(o)

        pend = jnp.where(full, 1, pend)
        return bf2, pend

    for (t, i, o, tl, d, V, cw, npw, nfull, tw, towner) in (
            (t0, i0, o0, tl0) + tabs[0],
            (t1, i1, o1, tl1) + tabs[1],
            (t2, i2, o2, tl2) + tabs[2]):
        pltpu.async_copy(i, idxb, sem_i).wait()
        lo_w = wid * (npw * cw)
        hi_w = jnp.minimum(V, lo_w + npw * cw)

        def select(q, pos, lo_w=lo_w, hi_w=hi_w):
            r = idxb[pl.ds(q * 16, 16)]
            m = jnp.logical_and(r >= lo_w, r < hi_w)
            packed = lax.shift_left(r - lo_w, 14) + (iota + q * 16)
            plsc.store_compressed(selb.at[pl.ds(pos, 16)], packed, mask=m)
            return pos + jnp.max(plsc.all_reduce_population_count(m))

        pos = lax.fori_loop(0, B // 16, select, jnp.int32(0), unroll=2)

        nch = jnp.minimum(jnp.maximum(nfull - wid * npw, 0), npw)

        def fire(ci, pm, t=t, d=d, cw=cw):
            start = pl.multiple_of(ci * cw, 128)
            return pltpu.async_copy(
                t.at[:, pl.ds(start, cw)],
                cbuf.at[pm, pl.ds(0, d), pl.ds(0, cw)], sem_c)

        def wait_chunk(t=t, d=d, cw=cw):
            pltpu.make_async_copy(
                t.at[:, pl.ds(0, cw)],
                cbuf.at[0, pl.ds(0, d), pl.ds(0, cw)], sem_c).wait()

        @pl.when(nch > 0)
        def _():
            fire(wid * npw, 0)

        @pl.when(nch > 1)
        def _():
            fire(wid * npw + 1, 1)

        def super_loop(s, carry, lo_w=lo_w, pos=pos, o=o, d=d, cw=cw,
                       npw=npw, nch=nch):
            sup_lo = s * (_SUP * cw)
            sup_hi = sup_lo + _SUP * cw

            def rescan(q, n, sup_lo=sup_lo, sup_hi=sup_hi, pos=pos):
                e = selb[pl.ds(q * 16, 16)]
                rl = lax.shift_right_logical(e, 14)
                m = jnp.logical_and(rl >= sup_lo, rl < sup_hi)
                m = jnp.logical_and(m, (q * 16 + iota) < pos)
                plsc.store_compressed(idxb.at[pl.ds(n, 16)], e, mask=m)
                return n + jnp.max(plsc.all_reduce_population_count(m))

            sup_n = lax.cond(
                s * _SUP < nch,
                lambda _: lax.fori_loop(0, lax.div(pos + 15, 16), rescan,
                                        jnp.int32(0)),
                lambda _: jnp.int32(0), 0)

            def chunk_loop(k, carry2, s=s, sup_n=sup_n, npw=npw, nch=nch,
                           o=o, cw=cw, d=d):
                il = s * _SUP + k
                ci = wid * npw + il

                @pl.when(il + 2 < nch)
                def _(il=il, ci=ci):
                    fire(ci + 2, lax.rem(il + 2, 3))

                def do(carry3, il=il, ci=ci, o=o, cw=cw, sup_n=sup_n, d=d):
                    wait_chunk()
                    pm = lax.rem(il, 3)
                    lo_c = il * cw

                    def group(g, c4, pm=pm, lo_c=lo_c, sup_n=sup_n, o=o,
                              cw=cw, d=d):
                        bf, pend = c4
                        e = idxb[pl.ds(g * 16, 16)]
                        rl = lax.shift_right_logical(e, 14)
                        j = lax.bitwise_and(e, 16383)
                        m = jnp.logical_and(rl >= lo_c, rl < lo_c + cw)
                        m = jnp.logical_and(m, (g * 16 + iota) < sup_n)
                        rv = jnp.where(m, rl - lo_c, 0)
                        jv = jnp.where(m, j, padrow)

                        def load_vals(rv=rv, pm=pm, d=d):
                            for c in range(d):
                                yield c, plsc.load_gather(
                                    cbuf,
                                    [jnp.full((16,), pm, jnp.int32),
                                     jnp.full((16,), c, jnp.int32), rv])

                        return group_step(o, bf, pend, m, jv, load_vals)

                    return lax.fori_loop(0, lax.div(sup_n + 15, 16),
                                         group, carry3)

                return lax.cond(il < nch, do, lambda c: c, carry2)

            return lax.fori_loop(0, _SUP, chunk_loop, carry)

        nsup = -(-npw // _SUP)
        carry = lax.fori_loop(0, nsup, super_loop,
                              (jnp.int32(0), jnp.int32(0)))
        bf, pend = carry

        if tw > 0:
            @pl.when(wid == towner)
            def _(tl=tl, o=o, d=d, tw=tw, bf=bf, pend=pend, pos=pos,
                  nfull=nfull, cw=cw, lo_w=lo_w):
                pltpu.async_copy(tl, cbuf.at[0, pl.ds(0, tw), pl.ds(0, 128)],
                                 sem_c).wait()
                lo_c = nfull * cw - lo_w

                def rescan(q, n, lo_c=lo_c, tw=tw, pos=pos):
                    e = selb[pl.ds(q * 16, 16)]
                    rl = lax.shift_right_logical(e, 14)
                    m = jnp.logical_and(rl >= lo_c, rl < lo_c + tw)
                    m = jnp.logical_and(m, (q * 16 + iota) < pos)
                    plsc.store_compressed(idxb.at[pl.ds(n, 16)], e, mask=m)
                    return n + jnp.max(plsc.all_reduce_population_count(m))

                n = lax.fori_loop(0, lax.div(pos + 15, 16), rescan,
                                  jnp.int32(0))

                def group(g, c4, n=n, o=o, d=d, lo_c=lo_c):
                    bf2, pend2 = c4
                    e = idxb[pl.ds(g * 16, 16)]
                    rl = lax.shift_right_logical(e, 14)
                    j = lax.bitwise_and(e, 16383)
                    m = (g * 16 + iota) < n
                    rv = jnp.where(m, rl - lo_c, 0)
                    jv = jnp.where(m, j, padrow)

                    def load_vals(rv=rv, d=d):
                        for c in range(d):
                            yield c, plsc.load_gather(
                                cbuf, [jnp.zeros((16,), jnp.int32), rv,
                                       jnp.full((16,), c, jnp.int32)])

                    return group_step(o, bf2, pend2, m, jv, load_vals)

                bf_t, pend_t = lax.fori_loop(0, lax.div(n + 15, 16),
                                             group, (bf, pend))
                _final(o, bf_t, pend_t, flush_blk, drain)

            @pl.when(wid != towner)
            def _(o=o, bf=bf, pend=pend):
                _final(o, bf, pend, flush_blk, drain)
        else:
            _final(o, bf, pend, flush_blk, drain)


def _final(o, bf, pend, flush_blk, drain):
    bfo = lax.rem(bf, _BATCH)

    @pl.when(bfo > 0)
    def _():
        flush_blk(o, bf, pend)
        drain(o)

    @pl.when(jnp.logical_and(bfo == 0, pend == 1))
    def _():
        drain(o)


def _chunk_plan(V, cw):
    nfull = V // cw
    tw = V - nfull * cw
    npw = -(-nfull // _NW)
    towner = nfull // npw if tw > 0 else 0
    return cw, npw, nfull, tw, towner


def _make_gather3(B, d0, V0, d1, V1, d2, V2):
    tabs = tuple((d, V) + _chunk_plan(V, cw)
                 for d, V, cw in ((d0, V0, 384), (d1, V1, 384),
                                  (d2, V2, 128)))
    max_cw = max(t[2] for t in tabs)
    mesh = plsc.VectorSubcoreMesh(core_axis_name="c", subcore_axis_name="s")
    return pl.kernel(
        functools.partial(_gather3_body, B, tabs),
        out_type=(
            jax.ShapeDtypeStruct((B + _NW, 128), jnp.float32),
            jax.ShapeDtypeStruct((B + _NW, 128), jnp.float32),
            jax.ShapeDtypeStruct((B + _NW, 128), jnp.float32),
        ),
        mesh=mesh,
        scratch_types=[
            pltpu.VMEM((B,), jnp.int32),
            pltpu.VMEM((B,), jnp.int32),
            pltpu.VMEM((3, 64, max_cw), jnp.float32),
            pltpu.VMEM((2, _BATCH, 128), jnp.float32),
            pltpu.VMEM((2, _BATCH), jnp.int32),
            pltpu.SemaphoreType.DMA,
            pltpu.SemaphoreType.DMA,
            pltpu.SemaphoreType.DMA,
            pltpu.SemaphoreType.DMA,
        ],
        compiler_params=pltpu.CompilerParams(needs_layout_passes=False),
    )


def _mlp_body(x_ref, w_ref, b_ref, o_ref):
    x = jnp.maximum(x_ref[...], 0.0)
    acc = jax.lax.dot_general(
        x, w_ref[...], (((1,), (0,)), ((), ())),
        preferred_element_type=jnp.float32)
    o_ref[...] = jnp.maximum(acc + b_ref[...], 0.0)


def _mlp(x, w, b):
    B, K = x.shape
    N = w.shape[1]
    BLK = 2048
    return pl.pallas_call(
        _mlp_body,
        grid=(B // BLK,),
        in_specs=[
            pl.BlockSpec((BLK, K), lambda i: (i, 0)),
            pl.BlockSpec((K, N), lambda i: (0, 0)),
            pl.BlockSpec((N,), lambda i: (0,)),
        ],
        out_specs=pl.BlockSpec((BLK, N), lambda i: (i, 0)),
        out_shape=jax.ShapeDtypeStruct((B, N), jnp.float32),
    )(x, w, b)


def _tail_pad(x, cw):
    nfull = x.shape[0] // cw
    d = x.shape[1]
    tail = x[nfull * cw:, :]
    return jnp.pad(tail, ((0, 0), (0, 128 - d)))


def kernel(arg0_1, arg1_1, arg2_1, arg3_1, arg4_1, arg5_1, arg6_1, arg7_1, arg8_1):
    B = arg1_1.shape[0]
    d0, d1, d2 = arg0_1.shape[1], arg2_1.shape[1], arg4_1.shape[1]
    g = _make_gather3(B, d0, arg0_1.shape[0], d1, arg2_1.shape[0],
                      d2, arg4_1.shape[0])
    f0, f1, f2 = g(arg0_1.T, arg1_1, arg2_1.T, arg3_1, arg4_1.T, arg5_1,
                   _tail_pad(arg0_1, 384), _tail_pad(arg2_1, 384),
                   _tail_pad(arg4_1, 128))
    relu_1 = _mlp(arg7_1, arg6_1, arg8_1)
    return (f0[:B, :d0], f1[:B, :d1], f2[:B, :d2], arg6_1, relu_1)


# speedup vs baseline: 1.4383x; 1.0209x over previous
import functools

import jax
import jax.numpy as jnp
from jax import lax
from jax.experimental import pallas as pl
from jax.experimental.pallas import tpu as pltpu
from jax.experimental.pallas import tpu_sc as plsc

_NC = 2
_NS = 16
_NW = _NC * _NS
_SUP = 2
_BATCH = 64


def _gather3_body(B, tabs,
                  t0, i0, t1, i1, t2, i2, tl0, tl1, tl2,
                  o0, o1, o2,
                  idxb, selb, cbuf, ext, jbig,
                  sem_i, sem_c, sem_s, sem_o):
    wid = lax.axis_index("s") * _NC + lax.axis_index("c")
    padrow = B + wid
    iota = lax.iota(jnp.int32, 16)

    def drain(o):
        pltpu.make_async_copy(
            ext.at[0], o.at[plsc.Indices(jbig.at[0])], sem_s).wait()

    def flush_blk(o, bf, pend):
        pm_b = lax.rem(lax.div(bf, _BATCH), 2)
        bfo = lax.rem(bf, _BATCH)

        def pad(g, _):
            rows = g * 16 + iota
            m = jnp.logical_and(rows >= bfo, rows < _BATCH)
            plsc.store_scatter(jbig, [jnp.full((16,), pm_b, jnp.int32),
                                      rows],
                               jnp.full((16,), padrow, jnp.int32), mask=m)
            return _

        lax.fori_loop(0, _BATCH // 16, pad, 0)
        pltpu.async_copy(ext.at[pm_b], o.at[plsc.Indices(jbig.at[pm_b])],
                         sem_s)

        @pl.when(pend == 1)
        def _():
            drain(o)

    def group_step(o, bf, pend, m, jv, load_vals):
        need = lax.rem(bf, _BATCH) > (_BATCH - 16)

        @pl.when(need)
        def _():
            flush_blk(o, bf, pend)

        bf = jnp.where(need, (lax.div(bf, _BATCH) + 1) * _BATCH, bf)
        pend = jnp.where(need, 1, pend)

        pm_b = lax.rem(lax.div(bf, _BATCH), 2)
        bfo = lax.rem(bf, _BATCH)
        mi = m.astype(jnp.int32)
        rank = plsc.cumsum(mi) - mi
        rows = bfo + rank
        pmbv = jnp.full((16,), pm_b, jnp.int32)
        for c, val in load_vals():
            plsc.store_scatter(ext, [pmbv, rows,
                                     jnp.full((16,), c, jnp.int32)],
                               val, mask=m)
        plsc.store_scatter(jbig, [pmbv, rows], jv, mask=m)
        cnt = jnp.max(plsc.all_reduce_population_count(m))
        bf2 = bf + cnt
        full = jnp.logical_and(lax.rem(bf2, _BATCH) == 0, cnt > 0)

        @pl.when(full)
        def _(pm_b=pm_b):
            pltpu.async_copy(ext.at[pm_b],
                             o.at[plsc.Indices(jbig.at[pm_b])], sem_s)

            @pl.when(pend == 1)
            def _():
                drain(o)

        pend = jnp.where(full, 1, pend)
        return bf2, pend

    for (t, i, o, tl, d, V, cw, npw, nfull, tw, towner) in (
            (t0, i0, o0, tl0) + tabs[0],
            (t1, i1, o1, tl1) + tabs[1],
            (t2, i2, o2, tl2) + tabs[2]):
        pltpu.async_copy(i, idxb, sem_i).wait()
        lo_w = wid * (npw * cw)
        hi_w = jnp.minimum(V, lo_w + npw * cw)

        def select(q, pos, lo_w=lo_w, hi_w=hi_w):
            r = idxb[pl.ds(q * 16, 16)]
            m = jnp.logical_and(r >= lo_w, r < hi_w)
            packed = lax.shift_left(r - lo_w, 14) + (iota + q * 16)
            plsc.store_compressed(selb.at[pl.ds(pos, 16)], packed, mask=m)
            return pos + jnp.max(plsc.all_reduce_population_count(m))

        pos = lax.fori_loop(0, B // 16, select, jnp.int32(0), unroll=2)

        nch = jnp.minimum(jnp.maximum(nfull - wid * npw, 0), npw)

        def fire(ci, pm, t=t, d=d, cw=cw):
            start = pl.multiple_of(ci * cw, 128)
            return pltpu.async_copy(
                t.at[:, pl.ds(start, cw)],
                cbuf.at[pm, pl.ds(0, d), pl.ds(0, cw)], sem_c)

        def wait_chunk(t=t, d=d, cw=cw):
            pltpu.make_async_copy(
                t.at[:, pl.ds(0, cw)],
                cbuf.at[0, pl.ds(0, d), pl.ds(0, cw)], sem_c).wait()

        @pl.when(nch > 0)
        def _():
            fire(wid * npw, 0)

        @pl.when(nch > 1)
        def _():
            fire(wid * npw + 1, 1)

        def super_loop(s, carry, lo_w=lo_w, pos=pos, o=o, d=d, cw=cw,
                       npw=npw, nch=nch):
            sup_lo = s * (_SUP * cw)
            sup_hi = sup_lo + _SUP * cw

            def rescan(q, n, sup_lo=sup_lo, sup_hi=sup_hi, pos=pos):
                e = selb[pl.ds(q * 16, 16)]
                rl = lax.shift_right_logical(e, 14)
                m = jnp.logical_and(rl >= sup_lo, rl < sup_hi)
                m = jnp.logical_and(m, (q * 16 + iota) < pos)
                plsc.store_compressed(idxb.at[pl.ds(n, 16)], e, mask=m)
                return n + jnp.max(plsc.all_reduce_population_count(m))

            sup_n = lax.cond(
                s * _SUP < nch,
                lambda _: lax.fori_loop(0, lax.div(pos + 15, 16), rescan,
                                        jnp.int32(0)),
                lambda _: jnp.int32(0), 0)

            def chunk_loop(k, carry2, s=s, sup_n=sup_n, npw=npw, nch=nch,
                           o=o, cw=cw, d=d):
                il = s * _SUP + k
                ci = wid * npw + il

                @pl.when(il + 2 < nch)
                def _(il=il, ci=ci):
                    fire(ci + 2, lax.rem(il + 2, 3))

                def do(carry3, il=il, ci=ci, o=o, cw=cw, sup_n=sup_n, d=d):
                    wait_chunk()
                    pm = lax.rem(il, 3)
                    lo_c = il * cw

                    def group(g, c4, pm=pm, lo_c=lo_c, sup_n=sup_n, o=o,
                              cw=cw, d=d):
                        bf, pend = c4
                        e = idxb[pl.ds(g * 16, 16)]
                        rl = lax.shift_right_logical(e, 14)
                        j = lax.bitwise_and(e, 16383)
                        m = jnp.logical_and(rl >= lo_c, rl < lo_c + cw)
                        m = jnp.logical_and(m, (g * 16 + iota) < sup_n)
                        rv = jnp.where(m, rl - lo_c, 0)
                        jv = jnp.where(m, j, padrow)

                        def load_vals(rv=rv, pm=pm, d=d):
                            for c in range(d):
                                yield c, plsc.load_gather(
                                    cbuf,
                                    [jnp.full((16,), pm, jnp.int32),
                                     jnp.full((16,), c, jnp.int32), rv])

                        return group_step(o, bf, pend, m, jv, load_vals)

                    return lax.fori_loop(0, lax.div(sup_n + 15, 16),
                                         group, carry3)

                return lax.cond(il < nch, do, lambda c: c, carry2)

            return lax.fori_loop(0, _SUP, chunk_loop, carry)

        nsup = -(-npw // _SUP)
        carry = lax.fori_loop(0, nsup, super_loop,
                              (jnp.int32(0), jnp.int32(0)))
        bf, pend = carry

        if tw > 0:
            @pl.when(wid == towner)
            def _(tl=tl, o=o, d=d, tw=tw, bf=bf, pend=pend, pos=pos,
                  nfull=nfull, cw=cw, lo_w=lo_w):
                pltpu.async_copy(tl, cbuf.at[0, pl.ds(0, tw), pl.ds(0, 128)],
                                 sem_c).wait()
                lo_c = nfull * cw - lo_w

                def rescan(q, n, lo_c=lo_c, tw=tw, pos=pos):
                    e = selb[pl.ds(q * 16, 16)]
                    rl = lax.shift_right_logical(e, 14)
                    m = jnp.logical_and(rl >= lo_c, rl < lo_c + tw)
                    m = jnp.logical_and(m, (q * 16 + iota) < pos)
                    plsc.store_compressed(idxb.at[pl.ds(n, 16)], e, mask=m)
                    return n + jnp.max(plsc.all_reduce_population_count(m))

                n = lax.fori_loop(0, lax.div(pos + 15, 16), rescan,
                                  jnp.int32(0))

                def group(g, c4, n=n, o=o, d=d, lo_c=lo_c):
                    bf2, pend2 = c4
                    e = idxb[pl.ds(g * 16, 16)]
                    rl = lax.shift_right_logical(e, 14)
                    j = lax.bitwise_and(e, 16383)
                    m = (g * 16 + iota) < n
                    rv = jnp.where(m, rl - lo_c, 0)
                    jv = jnp.where(m, j, padrow)

                    def load_vals(rv=rv, d=d):
                        for c in range(d):
                            yield c, plsc.load_gather(
                                cbuf, [jnp.zeros((16,), jnp.int32), rv,
                                       jnp.full((16,), c, jnp.int32)])

                    return group_step(o, bf2, pend2, m, jv, load_vals)

                bf_t, pend_t = lax.fori_loop(0, lax.div(n + 15, 16),
                                             group, (bf, pend))
                _final(o, bf_t, pend_t, flush_blk, drain)

            @pl.when(wid != towner)
            def _(o=o, bf=bf, pend=pend):
                _final(o, bf, pend, flush_blk, drain)
        else:
            _final(o, bf, pend, flush_blk, drain)


def _final(o, bf, pend, flush_blk, drain):
    bfo = lax.rem(bf, _BATCH)

    @pl.when(bfo > 0)
    def _():
        flush_blk(o, bf, pend)
        drain(o)

    @pl.when(jnp.logical_and(bfo == 0, pend == 1))
    def _():
        drain(o)


def _chunk_plan(V, cw):
    nfull = V // cw
    tw = V - nfull * cw
    npw = -(-nfull // _NW)
    towner = nfull // npw if tw > 0 else 0
    return cw, npw, nfull, tw, towner


def _make_gather3(B, d0, V0, d1, V1, d2, V2):
    tabs = tuple((d, V) + _chunk_plan(V, cw)
                 for d, V, cw in ((d0, V0, 384), (d1, V1, 384),
                                  (d2, V2, 128)))
    max_cw = max(t[2] for t in tabs)
    mesh = plsc.VectorSubcoreMesh(core_axis_name="c", subcore_axis_name="s")
    return pl.kernel(
        functools.partial(_gather3_body, B, tabs),
        out_type=(
            jax.ShapeDtypeStruct((B + _NW, 128), jnp.float32),
            jax.ShapeDtypeStruct((B + _NW, 128), jnp.float32),
            jax.ShapeDtypeStruct((B + _NW, 128), jnp.float32),
        ),
        mesh=mesh,
        scratch_types=[
            pltpu.VMEM((B,), jnp.int32),
            pltpu.VMEM((B,), jnp.int32),
            pltpu.VMEM((3, 64, max_cw), jnp.float32),
            pltpu.VMEM((2, _BATCH, 128), jnp.float32),
            pltpu.VMEM((2, _BATCH), jnp.int32),
            pltpu.SemaphoreType.DMA,
            pltpu.SemaphoreType.DMA,
            pltpu.SemaphoreType.DMA,
            pltpu.SemaphoreType.DMA,
        ],
        compiler_params=pltpu.CompilerParams(needs_layout_passes=False),
    )


def _mlp_body(x_ref, w_ref, b_ref, o_ref):
    x = jnp.maximum(x_ref[...], 0.0)
    acc = jax.lax.dot_general(
        x, w_ref[...], (((1,), (0,)), ((), ())),
        preferred_element_type=jnp.float32)
    o_ref[...] = jnp.maximum(acc + b_ref[...], 0.0)


def _mlp(x, w, b):
    B, K = x.shape
    N = w.shape[1]
    BLK = 2048
    return pl.pallas_call(
        _mlp_body,
        grid=(B // BLK,),
        in_specs=[
            pl.BlockSpec((BLK, K), lambda i: (i, 0)),
            pl.BlockSpec((K, N), lambda i: (0, 0)),
            pl.BlockSpec((N,), lambda i: (0,)),
        ],
        out_specs=pl.BlockSpec((BLK, N), lambda i: (i, 0)),
        out_shape=jax.ShapeDtypeStruct((B, N), jnp.float32),
    )(x, w, b)


def _tail_pad(x, cw):
    nfull = x.shape[0] // cw
    d = x.shape[1]
    tail = x[nfull * cw:, :]
    return jnp.pad(tail, ((0, 0), (0, 128 - d)))


def kernel(arg0_1, arg1_1, arg2_1, arg3_1, arg4_1, arg5_1, arg6_1, arg7_1, arg8_1):
    B = arg1_1.shape[0]
    d0, d1, d2 = arg0_1.shape[1], arg2_1.shape[1], arg4_1.shape[1]
    g = _make_gather3(B, d0, arg0_1.shape[0], d1, arg2_1.shape[0],
                      d2, arg4_1.shape[0])
    f0, f1, f2 = g(arg0_1.T, arg1_1, arg2_1.T, arg3_1, arg4_1.T, arg5_1,
                   _tail_pad(arg0_1, 384), _tail_pad(arg2_1, 384),
                   _tail_pad(arg4_1, 128))
    relu_1 = _mlp(arg7_1, arg6_1, arg8_1)
    return (f0[:B, :d0], f1[:B, :d1], f2[:B, :d2], arg6_1, relu_1)
